# baseline (speedup 1.0000x reference)
import functools

import jax
import jax.numpy as jnp
from jax import lax
from jax.experimental import pallas as pl
from jax.experimental.pallas import tpu as pltpu
from jax.experimental.pallas import tpu_sc as plsc

B = 16384
D = 100
F = 400

NC = 2
NS = 16
NW = NC * NS
BPW = B // NW
LANES = 16

_mesh = plsc.VectorSubcoreMesh(core_axis_name="c", subcore_axis_name="s")


@functools.partial(
    pl.kernel,
    mesh=_mesh,
    out_type=jax.ShapeDtypeStruct((B, D), jnp.float32),
    scratch_types=[
        pltpu.VMEM((BPW,), jnp.int32),
        pltpu.SemaphoreType.DMA,
    ],
)
def _gather(table_hbm, idx_hbm, out_hbm, idx_v, sem):
    wid = lax.axis_index("s") * NC + lax.axis_index("c")
    base = wid * BPW
    pltpu.sync_copy(idx_hbm.at[pl.ds(base, BPW)], idx_v)

    def fire_block(j, carry):
        vec = idx_v[pl.ds(j * LANES, LANES)]
        for l in range(LANES):
            r = vec[l]
            pltpu.make_async_copy(
                table_hbm.at[pl.ds(r, 1)],
                out_hbm.at[pl.ds(base + j * LANES + l, 1)],
                sem,
            ).start()
        return carry

    lax.fori_loop(0, BPW // LANES, fire_block, 0)
    pltpu.make_async_copy(
        table_hbm.at[pl.ds(0, BPW)], out_hbm.at[pl.ds(base, BPW)], sem
    ).wait()


BLK = 2048


def _mmt_body(x_ref, wt_ref, bt_ref, ot_ref):
    acc = jax.lax.dot_general(
        wt_ref[...], x_ref[...],
        (((1,), (1,)), ((), ())),
        preferred_element_type=jnp.float32,
    )
    ot_ref[...] = jnp.maximum(acc + bt_ref[...], 0.0)


def _project_t(x, wt, bt):
    return pl.pallas_call(
        _mmt_body,
        grid=(B // BLK,),
        in_specs=[
            pl.BlockSpec((BLK, D), lambda i: (i, 0)),
            pl.BlockSpec((F, D), lambda i: (0, 0)),
            pl.BlockSpec((F, 1), lambda i: (0, 0)),
        ],
        out_specs=pl.BlockSpec((F, BLK), lambda i: (0, i)),
        out_shape=jax.ShapeDtypeStruct((F, B), jnp.float32),
    )(x, wt, bt)


def kernel(inputs, table, W, b):
    idx = inputs.reshape(B).astype(jnp.int32)
    gathered = _gather(table, idx)
    out_t = _project_t(gathered, W.T, b.reshape(F, 1))
    return out_t.T


# speedup vs baseline: 4.0700x; 4.0700x over previous
import functools

import jax
import jax.numpy as jnp
from jax import lax
from jax.experimental import pallas as pl
from jax.experimental.pallas import tpu as pltpu
from jax.experimental.pallas import tpu_sc as plsc

B = 16384
D = 100
F = 400

NC = 2
NS = 16
NW = NC * NS
BPW = B // NW
LANES = 16

_mesh = plsc.VectorSubcoreMesh(core_axis_name="c", subcore_axis_name="s")


@functools.partial(
    pl.kernel,
    mesh=_mesh,
    out_type=jax.ShapeDtypeStruct((B, D), jnp.float32),
    scratch_types=[
        pltpu.VMEM((BPW,), jnp.int32),
        pltpu.VMEM((BPW, D), jnp.float32),
        pltpu.SemaphoreType.DMA,
    ],
)
def _gather(table_hbm, idx_hbm, out_hbm, idx_v, rows_v, sem):
    wid = lax.axis_index("s") * NC + lax.axis_index("c")
    base = wid * BPW
    pltpu.sync_copy(idx_hbm.at[pl.ds(base, BPW)], idx_v)

    def fire_block(j, carry):
        vec = idx_v[pl.ds(j * LANES, LANES)]
        for l in range(LANES):
            r = vec[l]
            pltpu.make_async_copy(
                table_hbm.at[pl.ds(r, 1)],
                rows_v.at[pl.ds(j * LANES + l, 1)],
                sem,
            ).start()
        return carry

    lax.fori_loop(0, BPW // LANES, fire_block, 0)
    pltpu.make_async_copy(table_hbm.at[pl.ds(0, BPW)], rows_v, sem).wait()
    pltpu.sync_copy(rows_v, out_hbm.at[pl.ds(base, BPW)])


BLK = 2048


def _mmt_body(x_ref, wt_ref, bt_ref, ot_ref):
    acc = jax.lax.dot_general(
        wt_ref[...], x_ref[...],
        (((1,), (1,)), ((), ())),
        preferred_element_type=jnp.float32,
    )
    ot_ref[...] = jnp.maximum(acc + bt_ref[...], 0.0)


def _project_t(x, wt, bt):
    return pl.pallas_call(
        _mmt_body,
        grid=(B // BLK,),
        in_specs=[
            pl.BlockSpec((BLK, D), lambda i: (i, 0)),
            pl.BlockSpec((F, D), lambda i: (0, 0)),
            pl.BlockSpec((F, 1), lambda i: (0, 0)),
        ],
        out_specs=pl.BlockSpec((F, BLK), lambda i: (0, i)),
        out_shape=jax.ShapeDtypeStruct((F, B), jnp.float32),
    )(x, wt, bt)


def kernel(inputs, table, W, b):
    idx = inputs.reshape(B).astype(jnp.int32)
    gathered = _gather(table, idx)
    out_t = _project_t(gathered, W.T, b.reshape(F, 1))
    return out_t.T


# speedup vs baseline: 4.1035x; 1.0082x over previous
import functools

import jax
import jax.numpy as jnp
from jax import lax
from jax.experimental import pallas as pl
from jax.experimental.pallas import tpu as pltpu
from jax.experimental.pallas import tpu_sc as plsc

B = 16384
D = 100
F = 400

NC = 2
NS = 16
NW = NC * NS
BPW = B // NW
LANES = 16

_mesh = plsc.VectorSubcoreMesh(core_axis_name="c", subcore_axis_name="s")


@functools.partial(
    pl.kernel,
    mesh=_mesh,
    out_type=jax.ShapeDtypeStruct((B, D), jnp.float32),
    scratch_types=[
        pltpu.VMEM((BPW,), jnp.int32),
        pltpu.VMEM((BPW, D), jnp.float32),
        pltpu.SemaphoreType.DMA,
    ],
)
def _gather(table_hbm, idx_hbm, out_hbm, idx_v, rows_v, sem):
    wid = lax.axis_index("s") * NC + lax.axis_index("c")
    base = wid * BPW
    pltpu.sync_copy(idx_hbm.at[pl.ds(base, BPW)], idx_v)

    def fire_block(j, carry):
        vec = idx_v[pl.ds(j * LANES, LANES)]
        for l in range(LANES):
            r = vec[l]
            pltpu.make_async_copy(
                table_hbm.at[pl.ds(r, 1)],
                rows_v.at[pl.ds(j * LANES + l, 1)],
                sem,
            ).start()
        return carry

    lax.fori_loop(0, BPW // LANES, fire_block, 0)
    pltpu.make_async_copy(table_hbm.at[pl.ds(0, BPW)], rows_v, sem).wait()
    pltpu.sync_copy(rows_v, out_hbm.at[pl.ds(base, BPW)])


BLK = 4096


def _mmt_body(x_ref, wt_ref, bt_ref, ot_ref):
    acc = jax.lax.dot_general(
        wt_ref[...], x_ref[...],
        (((1,), (1,)), ((), ())),
        preferred_element_type=jnp.float32,
    )
    ot_ref[...] = jnp.maximum(acc + bt_ref[...], 0.0)


def _project_t(x, wt, bt):
    return pl.pallas_call(
        _mmt_body,
        grid=(B // BLK,),
        in_specs=[
            pl.BlockSpec((BLK, D), lambda i: (i, 0)),
            pl.BlockSpec((F, D), lambda i: (0, 0)),
            pl.BlockSpec((F, 1), lambda i: (0, 0)),
        ],
        out_specs=pl.BlockSpec((F, BLK), lambda i: (0, i)),
        out_shape=jax.ShapeDtypeStruct((F, B), jnp.float32),
    )(x, wt, bt)


def kernel(inputs, table, W, b):
    idx = inputs.reshape(B).astype(jnp.int32)
    gathered = _gather(table, idx)
    out_t = _project_t(gathered, W.T, b.reshape(F, 1))
    return out_t.T
